# baseline (speedup 1.0000x reference)
import functools

import jax
import jax.numpy as jnp
from jax import lax
from jax.experimental import pallas as pl
from jax.experimental.pallas import tpu as pltpu
from jax.experimental.pallas import tpu_sc as plsc

_LANES = 16
_CL = 8
_NW = 32
_LS = 256
_BL = 256


def _sc_part(x, add_table, mult_table, ls):
    B, L, D = x.shape
    RW = ls // _NW
    JC = RW // _CL
    mesh = plsc.VectorSubcoreMesh(core_axis_name="core",
                                  subcore_axis_name="subcore")

    scratch = (
        [pltpu.VMEM((_CL, D), jnp.float32) for _ in range(2 * B)]
        + [pltpu.VMEM((_CL, D), jnp.float32) for _ in range(4)]
        + [pltpu.SemaphoreType.DMA] * (2 * B)
        + [pltpu.SemaphoreType.DMA] * 4
        + [pltpu.SemaphoreType.DMA] * (2 * B)
    )

    @functools.partial(
        pl.kernel,
        out_type=jax.ShapeDtypeStruct((B, ls, D), x.dtype),
        mesh=mesh,
        scratch_types=scratch,
    )
    def run(x_hbm, add_hbm, mult_hbm, o_hbm, *s):
        xb = s[0:2 * B]
        tb = s[2 * B:2 * B + 4]
        sx = s[2 * B + 4:4 * B + 4]
        st = s[4 * B + 4:4 * B + 8]
        so = s[4 * B + 8:6 * B + 8]
        wid = lax.axis_index("subcore") * 2 + lax.axis_index("core")
        l0 = wid * RW

        def start_in(j, ss):
            ls_ = pl.ds(l0 + j * _CL, _CL)
            cs = [pltpu.async_copy(x_hbm.at[b, ls_], xb[ss * B + b],
                                   sx[ss * B + b]) for b in range(B)]
            cs.append(pltpu.async_copy(add_hbm.at[ls_], tb[ss * 2 + 0],
                                       st[ss * 2 + 0]))
            cs.append(pltpu.async_copy(mult_hbm.at[ls_], tb[ss * 2 + 1],
                                       st[ss * 2 + 1]))
            return cs

        def start_out(j, ss):
            ls_ = pl.ds(l0 + j * _CL, _CL)
            return [pltpu.async_copy(xb[ss * B + b], o_hbm.at[b, ls_],
                                     so[ss * B + b]) for b in range(B)]

        ins = {0: start_in(0, 0)}
        outs = {}
        for j in range(JC):
            ss = j % 2
            if j + 1 < JC:
                if j - 1 >= 0:
                    for c in outs[j - 1]:
                        c.wait()
                ins[j + 1] = start_in(j + 1, (j + 1) % 2)
            for c in ins[j]:
                c.wait()

            @pl.loop(0, _CL)
            def _row(r, ss=ss):
                @pl.loop(0, D, step=_LANES, unroll=2)
                def _col(c, r=r, ss=ss):
                    sl = pl.ds(c, _LANES)
                    a = tb[ss * 2 + 0][r, sl]
                    m = tb[ss * 2 + 1][r, sl]
                    for b in range(B):
                        xb[ss * B + b][r, sl] = xb[ss * B + b][r, sl] * m + a

            outs[j] = start_out(j, ss)
        for j in (JC - 2, JC - 1):
            if j >= 0:
                for c in outs[j]:
                    c.wait()

    return run(x, add_table, mult_table)


def _tc_part(x, add_table, mult_table, ls):
    B, L, D = x.shape
    nb = ls // _BL

    def body(x_ref, add_ref, mult_ref, o_ref):
        o_ref[...] = x_ref[...] * mult_ref[...][None] + add_ref[...][None]

    grid = ((L - ls) // _BL, B)
    return pl.pallas_call(
        body,
        grid=grid,
        in_specs=[
            pl.BlockSpec((1, _BL, D), lambda i, b: (b, i + nb, 0)),
            pl.BlockSpec((_BL, D), lambda i, b: (i + nb, 0)),
            pl.BlockSpec((_BL, D), lambda i, b: (i + nb, 0)),
        ],
        out_specs=pl.BlockSpec((1, _BL, D), lambda i, b: (b, i + nb, 0)),
        out_shape=jax.ShapeDtypeStruct((B, L, D), x.dtype),
        compiler_params=pltpu.CompilerParams(
            dimension_semantics=("arbitrary", "arbitrary"),
        ),
    )(x, add_table, mult_table)


def kernel(x, add_table, mult_table):
    sc_out = _sc_part(x, add_table, mult_table, _LS)
    tc_full = _tc_part(x, add_table, mult_table, _LS)
    return lax.dynamic_update_slice(tc_full, sc_out, (0, 0, 0))


# speedup vs baseline: 1.0137x; 1.0137x over previous
import functools

import jax
import jax.numpy as jnp
from jax import lax
from jax.experimental import pallas as pl
from jax.experimental.pallas import tpu as pltpu
from jax.experimental.pallas import tpu_sc as plsc

_LANES = 16
_CL = 8
_NW = 32
_LS = 512
_BL = 256


def _sc_part(x, add_table, mult_table, ls):
    B, L, D = x.shape
    RW = ls // _NW
    JC = RW // _CL
    mesh = plsc.VectorSubcoreMesh(core_axis_name="core",
                                  subcore_axis_name="subcore")

    scratch = (
        [pltpu.VMEM((_CL, D), jnp.float32) for _ in range(2 * B)]
        + [pltpu.VMEM((_CL, D), jnp.float32) for _ in range(4)]
        + [pltpu.SemaphoreType.DMA] * (2 * B)
        + [pltpu.SemaphoreType.DMA] * 4
        + [pltpu.SemaphoreType.DMA] * (2 * B)
    )

    @functools.partial(
        pl.kernel,
        out_type=jax.ShapeDtypeStruct((B, ls, D), x.dtype),
        mesh=mesh,
        scratch_types=scratch,
    )
    def run(x_hbm, add_hbm, mult_hbm, o_hbm, *s):
        xb = s[0:2 * B]
        tb = s[2 * B:2 * B + 4]
        sx = s[2 * B + 4:4 * B + 4]
        st = s[4 * B + 4:4 * B + 8]
        so = s[4 * B + 8:6 * B + 8]
        wid = lax.axis_index("subcore") * 2 + lax.axis_index("core")
        l0 = wid * RW

        def start_in(j, ss):
            ls_ = pl.ds(l0 + j * _CL, _CL)
            cs = [pltpu.async_copy(x_hbm.at[b, ls_], xb[ss * B + b],
                                   sx[ss * B + b]) for b in range(B)]
            cs.append(pltpu.async_copy(add_hbm.at[ls_], tb[ss * 2 + 0],
                                       st[ss * 2 + 0]))
            cs.append(pltpu.async_copy(mult_hbm.at[ls_], tb[ss * 2 + 1],
                                       st[ss * 2 + 1]))
            return cs

        def start_out(j, ss):
            ls_ = pl.ds(l0 + j * _CL, _CL)
            return [pltpu.async_copy(xb[ss * B + b], o_hbm.at[b, ls_],
                                     so[ss * B + b]) for b in range(B)]

        ins = {0: start_in(0, 0)}
        outs = {}
        for j in range(JC):
            ss = j % 2
            if j + 1 < JC:
                if j - 1 >= 0:
                    for c in outs[j - 1]:
                        c.wait()
                ins[j + 1] = start_in(j + 1, (j + 1) % 2)
            for c in ins[j]:
                c.wait()

            @pl.loop(0, _CL)
            def _row(r, ss=ss):
                @pl.loop(0, D, step=_LANES, unroll=2)
                def _col(c, r=r, ss=ss):
                    sl = pl.ds(c, _LANES)
                    a = tb[ss * 2 + 0][r, sl]
                    m = tb[ss * 2 + 1][r, sl]
                    for b in range(B):
                        xb[ss * B + b][r, sl] = xb[ss * B + b][r, sl] * m + a

            outs[j] = start_out(j, ss)
        for j in (JC - 2, JC - 1):
            if j >= 0:
                for c in outs[j]:
                    c.wait()

    return run(x, add_table, mult_table)


def _tc_part(x, add_table, mult_table, ls):
    B, L, D = x.shape
    nb = ls // _BL

    def body(x_ref, add_ref, mult_ref, o_ref):
        o_ref[...] = x_ref[...] * mult_ref[...][None] + add_ref[...][None]

    grid = ((L - ls) // _BL, B)
    return pl.pallas_call(
        body,
        grid=grid,
        in_specs=[
            pl.BlockSpec((1, _BL, D), lambda i, b: (b, i + nb, 0)),
            pl.BlockSpec((_BL, D), lambda i, b: (i + nb, 0)),
            pl.BlockSpec((_BL, D), lambda i, b: (i + nb, 0)),
        ],
        out_specs=pl.BlockSpec((1, _BL, D), lambda i, b: (b, i + nb, 0)),
        out_shape=jax.ShapeDtypeStruct((B, L, D), x.dtype),
        compiler_params=pltpu.CompilerParams(
            dimension_semantics=("arbitrary", "arbitrary"),
        ),
    )(x, add_table, mult_table)


def kernel(x, add_table, mult_table):
    sc_out = _sc_part(x, add_table, mult_table, _LS)
    tc_full = _tc_part(x, add_table, mult_table, _LS)
    return lax.dynamic_update_slice(tc_full, sc_out, (0, 0, 0))
